# baseline (speedup 1.0000x reference)
import functools

import jax
import jax.numpy as jnp
from jax import lax
from jax.experimental import pallas as pl
from jax.experimental.pallas import tpu as pltpu
from jax.experimental.pallas import tpu_sc as plsc

_N, _D = 16384, 64
_NW = 32
_RPW = _N // _NW
_CH = 128
_NCH = _RPW // _CH
_R = 8
_NG = _CH // _R


def kernel(indicators, data, W, b):
    mesh = plsc.VectorSubcoreMesh(core_axis_name="c", subcore_axis_name="s")

    @functools.partial(
        pl.kernel,
        mesh=mesh,
        out_type=jax.ShapeDtypeStruct((_N, _D), jnp.float32),
        scratch_types=[
            pltpu.VMEM((_RPW + 16,), jnp.int32),
            pltpu.VMEM((_CH, _D), jnp.float32),
            pltpu.VMEM((_D, _D), jnp.float32),
            pltpu.VMEM((_D,), jnp.float32),
            pltpu.VMEM((_CH, _D), jnp.float32),
        ],
    )
    def k(ind_hbm, data_hbm, w_hbm, b_hbm, out_hbm, ind_v, data_v, w_v, b_v, out_v):
        wid = lax.axis_index("s") * 2 + lax.axis_index("c")
        base = wid * _RPW
        pltpu.sync_copy(ind_hbm.at[pl.ds(base, _RPW)], ind_v.at[pl.ds(0, _RPW)])
        pltpu.sync_copy(w_hbm, w_v)
        pltpu.sync_copy(b_hbm, b_v)

        bias = [b_v[pl.ds(16 * c, 16)] for c in range(4)]

        def chunk_body(ch, carry0):
            pltpu.sync_copy(data_hbm.at[pl.ds(base + ch * _CH, _CH)], data_v)

            def group_body(g, carry):
                ivec = ind_v[pl.ds(ch * _CH + g * _R, 16)]
                rowv = [
                    [data_v[g * _R + r, pl.ds(16 * c, 16)] for c in range(4)]
                    for r in range(_R)
                ]
                accs = [[bias[c] for c in range(4)] for _ in range(_R)]
                for kk in range(_D):
                    wvec = [w_v[kk, pl.ds(16 * c, 16)] for c in range(4)]
                    for r in range(_R):
                        s = rowv[r][kk // 16][kk % 16]
                        for c in range(4):
                            accs[r][c] = accs[r][c] + s * wvec[c]
                for r in range(_R):
                    mf = jnp.where(ivec[r] != 0, 1.0, 0.0)
                    for c in range(4):
                        out_v[g * _R + r, pl.ds(16 * c, 16)] = accs[r][c] * mf
                return carry

            lax.fori_loop(0, _NG, group_body, 0)
            pltpu.sync_copy(out_v, out_hbm.at[pl.ds(base + ch * _CH, _CH)])
            return carry0

        lax.fori_loop(0, _NCH, chunk_body, 0)

    return k(indicators, data, W, b)


# speedup vs baseline: 7.4596x; 7.4596x over previous
import jax
import jax.numpy as jnp
from jax.experimental import pallas as pl
from jax.experimental.pallas import tpu as pltpu

_N, _D = 16384, 64
_NC = 8
_CH = _N // _NC


def _body(ind_hbm, x_hbm, w_hbm, b_hbm, o_hbm,
          ind_v, w_v, b_v, xbuf, obuf,
          insem, outsem, csem):
    pltpu.make_async_copy(w_hbm, w_v, csem.at[0]).start()
    pltpu.make_async_copy(b_hbm, b_v, csem.at[1]).start()
    pltpu.make_async_copy(ind_hbm, ind_v, csem.at[2]).start()
    for c in range(_NC):
        pltpu.make_async_copy(
            x_hbm.at[pl.ds(c * _CH, _CH)], xbuf.at[c], insem.at[c]
        ).start()
    pltpu.make_async_copy(w_hbm, w_v, csem.at[0]).wait()
    pltpu.make_async_copy(b_hbm, b_v, csem.at[1]).wait()
    pltpu.make_async_copy(ind_hbm, ind_v, csem.at[2]).wait()
    w = w_v[...]
    bias = b_v[...]
    for c in range(_NC):
        pltpu.make_async_copy(
            x_hbm.at[pl.ds(c * _CH, _CH)], xbuf.at[c], insem.at[c]
        ).wait()
        acc = jnp.dot(xbuf[c], w, preferred_element_type=jnp.float32) + bias
        mrow = jnp.where(ind_v[:, pl.ds(c * _CH, _CH)] != 0, 1.0, 0.0)
        obuf[c] = acc * jnp.transpose(mrow)
        pltpu.make_async_copy(
            obuf.at[c], o_hbm.at[pl.ds(c * _CH, _CH)], outsem.at[c]
        ).start()
    for c in range(_NC):
        pltpu.make_async_copy(
            obuf.at[c], o_hbm.at[pl.ds(c * _CH, _CH)], outsem.at[c]
        ).wait()


def kernel(indicators, data, W, b):
    N, D = data.shape
    return pl.pallas_call(
        _body,
        in_specs=[
            pl.BlockSpec(memory_space=pl.ANY),
            pl.BlockSpec(memory_space=pl.ANY),
            pl.BlockSpec(memory_space=pl.ANY),
            pl.BlockSpec(memory_space=pl.ANY),
        ],
        out_specs=pl.BlockSpec(memory_space=pl.ANY),
        out_shape=jax.ShapeDtypeStruct((N, D), jnp.float32),
        scratch_shapes=[
            pltpu.VMEM((1, _N), jnp.int32),
            pltpu.VMEM((D, D), jnp.float32),
            pltpu.VMEM((1, D), jnp.float32),
            pltpu.VMEM((_NC, _CH, _D), jnp.float32),
            pltpu.VMEM((_NC, _CH, _D), jnp.float32),
            pltpu.SemaphoreType.DMA((_NC,)),
            pltpu.SemaphoreType.DMA((_NC,)),
            pltpu.SemaphoreType.DMA((3,)),
        ],
    )(indicators.reshape(1, N), data, W, b.reshape(1, D))
